# baseline (speedup 1.0000x reference)
import functools

import jax
import jax.numpy as jnp
import numpy as np
from jax import lax
from jax.experimental import pallas as pl
from jax.experimental.pallas import tpu as pltpu
from jax.experimental.pallas import tpu_sc as plsc

N = 10000
E = 320000
D = 128
NUM_BASIS = 16

NW = 32
EPW = E // NW
C = 80
NCH = EPW // C
NBLK = 5
CPB = NCH // NBLK

RPT = 640
LAST = N - 15 * RPT

BE = 6400
BN = 2000


def _radial_body(rbft_ref, w1_ref, b1_ref, w2_ref, b2_ref, out_ref):
    h = jax.lax.dot_general(
        rbft_ref[...], w1_ref[...], (((0,), (0,)), ((), ())),
        preferred_element_type=jnp.float32)
    h = h + b1_ref[...]
    h = h * jax.nn.sigmoid(h)
    w = jnp.dot(h, w2_ref[...], preferred_element_type=jnp.float32) + b2_ref[...]
    out_ref[...] = w.astype(jnp.bfloat16)


def _radial(edge_rbf, W1, b1, W2, b2):
    grid = (E // BE,)
    return pl.pallas_call(
        _radial_body,
        grid=grid,
        in_specs=[
            pl.BlockSpec((NUM_BASIS, BE), lambda i: (0, i)),
            pl.BlockSpec((NUM_BASIS, D), lambda i: (0, 0)),
            pl.BlockSpec((1, D), lambda i: (0, 0)),
            pl.BlockSpec((D, D), lambda i: (0, 0)),
            pl.BlockSpec((1, D), lambda i: (0, 0)),
        ],
        out_specs=pl.BlockSpec((BE, D), lambda i: (i, 0)),
        out_shape=jax.ShapeDtypeStruct((E, D), jnp.bfloat16),
    )(edge_rbf.T, W1, b1.reshape(1, D), W2, b2.reshape(1, D))


def _sc_body(col_hbm, row_hbm, nf_hbm, w_hbm, zeros_hbm, out_hbm,
             col_v, row_v, rows_v, wts_v, msg_v, agg_sh, sem0):
    cid = lax.axis_index("c")
    sid = lax.axis_index("s")
    wid = cid * 16 + sid

    @pl.when(sid < 15)
    def _():
        pltpu.sync_copy(zeros_hbm, agg_sh.at[pl.ds(sid * RPT, RPT)])

    @pl.when(sid == 15)
    def _():
        pltpu.sync_copy(zeros_hbm.at[pl.ds(0, LAST)],
                        agg_sh.at[pl.ds(15 * RPT, LAST)])

    plsc.subcore_barrier()

    def block_body(b, carry):
        pltpu.sync_copy(col_hbm.at[wid, b], col_v)
        pltpu.sync_copy(row_hbm.at[wid, b], row_v)

        wbase = wid * EPW + b * CPB * C

        def chunk_body(t, c1):
            gcopy = pltpu.async_copy(nf_hbm.at[col_v.at[t]], rows_v, sem0)
            woff = pl.multiple_of((wbase + t * C) // 2, 8)
            pltpu.sync_copy(w_hbm.at[pl.ds(woff, C // 2)], wts_v)
            gcopy.wait()

            himask = jnp.full((16,), -65536, jnp.int32)

            def pair_body(e2, c2):
                for half in range(2):
                    e = 2 * e2 + half
                    for q in range(D // 32):
                        w = wts_v[e2, pl.ds(half * 64 + q * 16, 16)]
                        wlo = lax.bitcast_convert_type(w << 16, jnp.float32)
                        whi = lax.bitcast_convert_type(w & himask, jnp.float32)
                        slo = pl.ds(q * 32, 16)
                        shi = pl.ds(q * 32 + 16, 16)
                        msg_v[e, slo] = rows_v[e, slo] * wlo
                        msg_v[e, shi] = rows_v[e, shi] * whi
                return c2

            lax.fori_loop(0, C // 2, pair_body, 0)
            pltpu.sync_copy(msg_v, agg_sh.at[row_v.at[t]], add=True)
            return c1

        lax.fori_loop(0, CPB, chunk_body, 0)
        return carry

    lax.fori_loop(0, NBLK, block_body, 0)
    plsc.subcore_barrier()

    @pl.when(sid < 15)
    def _():
        pltpu.sync_copy(agg_sh.at[pl.ds(sid * RPT, RPT)],
                        out_hbm.at[cid, pl.ds(sid * RPT, RPT)])

    @pl.when(sid == 15)
    def _():
        pltpu.sync_copy(agg_sh.at[pl.ds(15 * RPT, LAST)],
                        out_hbm.at[cid, pl.ds(15 * RPT, LAST)])


def _sc_aggregate(col_r, row_r, node_feats, weights, zeros):
    mesh = plsc.VectorSubcoreMesh(core_axis_name="c", subcore_axis_name="s")
    f = functools.partial(
        pl.kernel,
        mesh=mesh,
        out_type=jax.ShapeDtypeStruct((2, N, D), jnp.float32),
        scratch_types=[
            pltpu.VMEM((CPB, C), jnp.int32),
            pltpu.VMEM((CPB, C), jnp.int32),
            pltpu.VMEM((C, D), jnp.float32),
            pltpu.VMEM((C // 2, D), jnp.int32),
            pltpu.VMEM((C, D), jnp.float32),
            pltpu.VMEM_SHARED((N, D), jnp.float32),
            pltpu.SemaphoreType.DMA,
        ],
    )(_sc_body)
    return f(col_r, row_r, node_feats, weights, zeros)


def _epilogue_body(nf_ref, p_ref, wself_ref, bself_ref, wc1_ref, wc2_ref,
                   bcat_ref, out_ref):
    nf = nf_ref[...]
    self_out = (
        jnp.dot(nf, wself_ref[...], preferred_element_type=jnp.float32)
        + bself_ref[...]
    )
    agg = p_ref[0] + p_ref[1]
    out_ref[...] = (
        jnp.dot(self_out, wc1_ref[...], preferred_element_type=jnp.float32)
        + jnp.dot(agg, wc2_ref[...], preferred_element_type=jnp.float32)
        + bcat_ref[...]
    )


def _epilogue(node_feats, partial, W_self, b_self, wc1, wc2, b_cat):
    grid = (N // BN,)
    return pl.pallas_call(
        _epilogue_body,
        grid=grid,
        in_specs=[
            pl.BlockSpec((BN, D), lambda i: (i, 0)),
            pl.BlockSpec((2, BN, D), lambda i: (0, i, 0)),
            pl.BlockSpec((D, D), lambda i: (0, 0)),
            pl.BlockSpec((1, D), lambda i: (0, 0)),
            pl.BlockSpec((D, D), lambda i: (0, 0)),
            pl.BlockSpec((D, D), lambda i: (0, 0)),
            pl.BlockSpec((1, D), lambda i: (0, 0)),
        ],
        out_specs=pl.BlockSpec((BN, D), lambda i: (i, 0)),
        out_shape=jax.ShapeDtypeStruct((N, D), jnp.float32),
    )(node_feats, partial, W_self, b_self.reshape(1, D), wc1, wc2,
      b_cat.reshape(1, D))


_PERM = np.concatenate(
    [q * 32 + np.concatenate([np.arange(0, 32, 2), np.arange(1, 32, 2)])
     for q in range(D // 32)])


def kernel(node_feats, edge_index, edge_rbf, edge_sh, W1, b1, W2, b2,
           W_self, b_self, W_cat, b_cat):
    del edge_sh
    weights = _radial(edge_rbf, W1, b1, W2, b2)
    row_r = edge_index[0].astype(jnp.int32).reshape(NW, NBLK, CPB, C)
    col_r = edge_index[1].astype(jnp.int32).reshape(NW, NBLK, CPB, C)
    nf_p = node_feats[:, _PERM]
    w_pk = jax.lax.bitcast_convert_type(
        weights.reshape(E // 2, D, 2), jnp.int32)
    zeros = jnp.zeros((RPT, D), jnp.float32)
    partial = _sc_aggregate(col_r, row_r, nf_p, w_pk, zeros)
    wc2 = W_cat[D:][_PERM]
    return _epilogue(node_feats, partial, W_self, b_self, W_cat[:D], wc2,
                     b_cat)


# speedup vs baseline: 30.4436x; 30.4436x over previous
import functools

import jax
import jax.numpy as jnp
from jax import lax
from jax.experimental import pallas as pl
from jax.experimental.pallas import tpu as pltpu
from jax.experimental.pallas import tpu_sc as plsc

N = 10000
E = 320000
D = 128
NUM_BASIS = 16

NW = 32
EPW = E // NW
C = 80
NCH = EPW // C
NBLK = 5
CPB = NCH // NBLK

RPT = 640
LAST = N - 15 * RPT

BE = 6400
BN = 2000


def _radial_body(rbft_ref, w1_ref, b1_ref, w2_ref, b2_ref, out_ref):
    h = jax.lax.dot_general(
        rbft_ref[...], w1_ref[...], (((0,), (0,)), ((), ())),
        preferred_element_type=jnp.float32)
    h = h + b1_ref[...]
    h = h * jax.nn.sigmoid(h)
    out_ref[...] = (
        jnp.dot(h, w2_ref[...], preferred_element_type=jnp.float32) + b2_ref[...]
    )


def _radial(edge_rbf, W1, b1, W2, b2):
    grid = (E // BE,)
    return pl.pallas_call(
        _radial_body,
        grid=grid,
        in_specs=[
            pl.BlockSpec((NUM_BASIS, BE), lambda i: (0, i)),
            pl.BlockSpec((NUM_BASIS, D), lambda i: (0, 0)),
            pl.BlockSpec((1, D), lambda i: (0, 0)),
            pl.BlockSpec((D, D), lambda i: (0, 0)),
            pl.BlockSpec((1, D), lambda i: (0, 0)),
        ],
        out_specs=pl.BlockSpec((BE, D), lambda i: (i, 0)),
        out_shape=jax.ShapeDtypeStruct((E, D), jnp.float32),
    )(edge_rbf.T, W1, b1.reshape(1, D), W2, b2.reshape(1, D))


def _sc_body(col_hbm, row_hbm, nf_hbm, w_hbm, zeros_hbm, out_hbm,
             col_v, row_v, rows0_v, rows1_v, wts_v, agg_sh, sem0, sem1):
    cid = lax.axis_index("c")
    sid = lax.axis_index("s")
    wid = cid * 16 + sid

    @pl.when(sid < 15)
    def _():
        pltpu.sync_copy(zeros_hbm, agg_sh.at[pl.ds(sid * RPT, RPT)])

    @pl.when(sid == 15)
    def _():
        pltpu.sync_copy(zeros_hbm.at[pl.ds(0, LAST)],
                        agg_sh.at[pl.ds(15 * RPT, LAST)])

    plsc.subcore_barrier()

    bufs = (rows0_v, rows1_v)
    sems = (sem0, sem1)

    def block_body(b, carry):
        pltpu.sync_copy(col_hbm.at[wid, b], col_v)
        pltpu.sync_copy(row_hbm.at[wid, b], row_v)

        wbase = wid * EPW + b * CPB * C

        def chunk_body(t, c1):
            gcopy = pltpu.async_copy(nf_hbm.at[col_v.at[t]], rows0_v, sem0)
            pltpu.sync_copy(w_hbm.at[pl.ds(wbase + t * C, C)], wts_v)
            gcopy.wait()

            def edge_body(e, c2):
                for q in range(D // 16):
                    s = pl.ds(q * 16, 16)
                    rows0_v[e, s] = rows0_v[e, s] * wts_v[e, s]
                return c2

            lax.fori_loop(0, C, edge_body, 0)
            pltpu.sync_copy(rows0_v, agg_sh.at[row_v.at[t]], add=True)
            return c1

        lax.fori_loop(0, CPB, chunk_body, 0)
        return carry

    lax.fori_loop(0, NBLK, block_body, 0)
    plsc.subcore_barrier()

    @pl.when(sid < 15)
    def _():
        pltpu.sync_copy(agg_sh.at[pl.ds(sid * RPT, RPT)],
                        out_hbm.at[cid, pl.ds(sid * RPT, RPT)])

    @pl.when(sid == 15)
    def _():
        pltpu.sync_copy(agg_sh.at[pl.ds(15 * RPT, LAST)],
                        out_hbm.at[cid, pl.ds(15 * RPT, LAST)])


def _sc_aggregate(col_r, row_r, node_feats, weights, zeros):
    mesh = plsc.VectorSubcoreMesh(core_axis_name="c", subcore_axis_name="s")
    f = functools.partial(
        pl.kernel,
        mesh=mesh,
        out_type=jax.ShapeDtypeStruct((2, N, D), jnp.float32),
        scratch_types=[
            pltpu.VMEM((CPB, C), jnp.int32),
            pltpu.VMEM((CPB, C), jnp.int32),
            pltpu.VMEM((C, D), jnp.float32),
            pltpu.VMEM((C, D), jnp.float32),
            pltpu.VMEM((C, D), jnp.float32),
            pltpu.VMEM_SHARED((N, D), jnp.float32),
            pltpu.SemaphoreType.DMA,
            pltpu.SemaphoreType.DMA,
        ],
    )(_sc_body)
    return f(col_r, row_r, node_feats, weights, zeros)


def _epilogue_body(nf_ref, p_ref, wself_ref, bself_ref, wc1_ref, wc2_ref,
                   bcat_ref, out_ref):
    nf = nf_ref[...]
    self_out = (
        jnp.dot(nf, wself_ref[...], preferred_element_type=jnp.float32)
        + bself_ref[...]
    )
    agg = p_ref[0] + p_ref[1]
    out_ref[...] = (
        jnp.dot(self_out, wc1_ref[...], preferred_element_type=jnp.float32)
        + jnp.dot(agg, wc2_ref[...], preferred_element_type=jnp.float32)
        + bcat_ref[...]
    )


def _epilogue(node_feats, partial, W_self, b_self, W_cat, b_cat):
    grid = (N // BN,)
    return pl.pallas_call(
        _epilogue_body,
        grid=grid,
        in_specs=[
            pl.BlockSpec((BN, D), lambda i: (i, 0)),
            pl.BlockSpec((2, BN, D), lambda i: (0, i, 0)),
            pl.BlockSpec((D, D), lambda i: (0, 0)),
            pl.BlockSpec((1, D), lambda i: (0, 0)),
            pl.BlockSpec((D, D), lambda i: (0, 0)),
            pl.BlockSpec((D, D), lambda i: (0, 0)),
            pl.BlockSpec((1, D), lambda i: (0, 0)),
        ],
        out_specs=pl.BlockSpec((BN, D), lambda i: (i, 0)),
        out_shape=jax.ShapeDtypeStruct((N, D), jnp.float32),
    )(node_feats, partial, W_self, b_self.reshape(1, D), W_cat[:D],
      W_cat[D:], b_cat.reshape(1, D))


def kernel(node_feats, edge_index, edge_rbf, edge_sh, W1, b1, W2, b2,
           W_self, b_self, W_cat, b_cat):
    del edge_sh
    weights = _radial(edge_rbf, W1, b1, W2, b2)
    row_r = edge_index[0].astype(jnp.int32).reshape(NW, NBLK, CPB, C)
    col_r = edge_index[1].astype(jnp.int32).reshape(NW, NBLK, CPB, C)
    zeros = jnp.zeros((RPT, D), jnp.float32)
    partial = _sc_aggregate(col_r, row_r, node_feats, weights, zeros)
    return _epilogue(node_feats, partial, W_self, b_self, W_cat, b_cat)
